# baseline (speedup 1.0000x reference)
import functools

import jax
import jax.numpy as jnp
from jax import lax
from jax.experimental import pallas as pl
from jax.experimental.pallas import tpu as pltpu
from jax.experimental.pallas import tpu_sc as plsc

_VOCAB = 1000000
_L = 200
_B = 4096
_D = 64
_TOPICS = 50
_PAD_T = 128

_NC = 2
_NS = 16
_NW = _NC * _NS
_SPW = _B // _NW
_LANES = 16

_VB = 128
_NFULL = _VOCAB // _VB
_TAIL = _VOCAB - _NFULL * _VB
_NIT = 246

_CH0 = 128
_CH1 = _L - _CH0


def _tr_body(tt_hbm, tail_hbm, out_hbm, in0, in1, ob0, ob1, si0, si1, so0, so1):
    wid = lax.axis_index("s") * _NC + lax.axis_index("c")
    inbufs = (in0, in1)
    obufs = (ob0, ob1)
    sins = (si0, si1)
    souts = (so0, so1)
    rows = [lax.iota(jnp.int32, 16) + 16 * e for e in range(_D // _LANES)]

    @pl.when(wid == 0)
    def _():
        pltpu.sync_copy(tail_hbm, out_hbm.at[pl.ds(_NFULL * _VB * _D, _TAIL * _D)])

    def blk(it):
        return wid + _NW * it

    def start_in(it, which):
        b = blk(it)

        @pl.when(b < _NFULL)
        def _():
            c0 = pl.multiple_of(b * _VB, _VB)
            pltpu.make_async_copy(
                tt_hbm.at[:, pl.ds(c0, _VB)], inbufs[which], sins[which]).start()

    def wait_in(it, which):
        b = blk(it)

        @pl.when(b < _NFULL)
        def _():
            pltpu.make_async_copy(
                tt_hbm.at[:, pl.ds(0, _VB)], inbufs[which], sins[which]).wait()

    def extract(it, which):
        b = blk(it)
        inb = inbufs[which]
        obuf = obufs[which]

        def jbody(p, _):
            for u in range(2):
                j = 2 * p + u
                col = rows[0] * 0 + j
                base = pl.multiple_of(j * _D, 8)
                for e in range(_D // _LANES):
                    v = plsc.load_gather(inb, [rows[e], col])
                    obuf[pl.ds(base + 16 * e, 16)] = v
            return 0

        @pl.when(b < _NFULL)
        def _():
            lax.fori_loop(0, _VB // 2, jbody, 0)

    def start_out(it, which):
        b = blk(it)

        @pl.when(b < _NFULL)
        def _():
            o0 = pl.multiple_of(b * (_VB * _D), _VB * _D)
            pltpu.make_async_copy(
                obufs[which], out_hbm.at[pl.ds(o0, _VB * _D)], souts[which]).start()

    def wait_out(it, which):
        b = blk(it)

        @pl.when(b < _NFULL)
        def _():
            pltpu.make_async_copy(
                obufs[which], out_hbm.at[pl.ds(0, _VB * _D)], souts[which]).wait()

    start_in(0, 0)

    def pair(p, _):
        i0 = 2 * p
        start_in(i0 + 1, 1)
        wait_in(i0, 0)

        @pl.when(p > 0)
        def _():
            wait_out(i0 - 2, 0)

        extract(i0, 0)
        start_out(i0, 0)
        start_in(i0 + 2, 0)
        wait_in(i0 + 1, 1)

        @pl.when(p > 0)
        def _():
            wait_out(i0 - 1, 1)

        extract(i0 + 1, 1)
        start_out(i0 + 1, 1)
        return 0

    lax.fori_loop(0, _NIT // 2, pair, 0)
    wait_out(_NIT - 2, 0)
    wait_out(_NIT - 1, 1)


@functools.cache
def _get_transpose():
    return pl.kernel(
        _tr_body,
        out_type=jax.ShapeDtypeStruct((_VOCAB * _D,), jnp.float32),
        mesh=plsc.VectorSubcoreMesh(core_axis_name="c", subcore_axis_name="s"),
        scratch_types=[
            pltpu.VMEM((_D, _VB), jnp.float32),
            pltpu.VMEM((_D, _VB), jnp.float32),
            pltpu.VMEM((_VB * _D,), jnp.float32),
            pltpu.VMEM((_VB * _D,), jnp.float32),
            pltpu.SemaphoreType.DMA,
            pltpu.SemaphoreType.DMA,
            pltpu.SemaphoreType.DMA,
            pltpu.SemaphoreType.DMA,
        ],
        compiler_params=pltpu.CompilerParams(needs_layout_passes=False),
    )


def _pool_body(idx_hbm, table_hbm, out_hbm, idx_v, buf0, buf1, sums_v, sem0, sem1):
    wid = lax.axis_index("s") * _NC + lax.axis_index("c")
    base = wid * _SPW
    pltpu.sync_copy(idx_hbm.at[pl.ds(base * _L, _SPW * _L)], idx_v)

    bufs = (buf0, buf1)
    sems = (sem0, sem1)

    def start_gather(s, which):
        buf = bufs[which]
        sem = sems[which]
        off = pl.multiple_of(s * _L, 8)
        c0 = pltpu.make_async_copy(
            table_hbm.at[idx_v.at[pl.ds(off, _CH0)]], buf.at[pl.ds(0, _CH0)], sem)
        c1 = pltpu.make_async_copy(
            table_hbm.at[idx_v.at[pl.ds(off + _CH0, _CH1)]], buf.at[pl.ds(_CH0, _CH1)], sem)
        c0.start()
        c1.start()

    def wait_gather(which):
        buf = bufs[which]
        sem = sems[which]
        pltpu.make_async_copy(
            table_hbm.at[idx_v.at[pl.ds(0, _CH0)]], buf.at[pl.ds(0, _CH0)], sem).wait()
        pltpu.make_async_copy(
            table_hbm.at[idx_v.at[pl.ds(0, _CH1)]], buf.at[pl.ds(_CH0, _CH1)], sem).wait()

    def accumulate(s, which):
        buf = bufs[which]

        def rbody(r, accs):
            return tuple(a + buf[r, pl.ds(c * _LANES, _LANES)]
                         for c, a in enumerate(accs))

        accs = lax.fori_loop(
            0, _L, rbody,
            tuple(jnp.zeros((_LANES,), jnp.float32) for _ in range(_D // _LANES)))
        for c, a in enumerate(accs):
            sums_v[s, pl.ds(c * _LANES, _LANES)] = a * (1.0 / _L)

    start_gather(0, 0)

    def pair_body(p, _):
        s0 = p * 2
        start_gather(s0 + 1, 1)
        wait_gather(0)
        accumulate(s0, 0)

        @pl.when(s0 + 2 < _SPW)
        def _():
            start_gather(s0 + 2, 0)

        wait_gather(1)
        accumulate(s0 + 1, 1)
        return 0

    lax.fori_loop(0, _SPW // 2, pair_body, 0)
    pltpu.sync_copy(sums_v, out_hbm.at[pl.ds(base, _SPW)])


@functools.cache
def _get_pool():
    return pl.kernel(
        _pool_body,
        out_type=jax.ShapeDtypeStruct((_B, _D), jnp.float32),
        mesh=plsc.VectorSubcoreMesh(core_axis_name="c", subcore_axis_name="s"),
        scratch_types=[
            pltpu.VMEM((_SPW * _L,), jnp.int32),
            pltpu.VMEM((_L, _D), jnp.float32),
            pltpu.VMEM((_L, _D), jnp.float32),
            pltpu.VMEM((_SPW, _D), jnp.float32),
            pltpu.SemaphoreType.DMA,
            pltpu.SemaphoreType.DMA,
        ],
        compiler_params=pltpu.CompilerParams(use_tc_tiling_on_sc=False),
    )


def _mlp_body(x_ref, w1_ref, b1_ref, w2_ref, b2_ref, wc_ref, bc_ref, out_ref):
    x = x_ref[...]
    h = jnp.maximum(jnp.dot(x, w1_ref[...],
                            preferred_element_type=jnp.float32) + b1_ref[...], 0.0)
    h = jnp.maximum(jnp.dot(h, w2_ref[...],
                            preferred_element_type=jnp.float32) + b2_ref[...], 0.0)
    z = jnp.dot(h, wc_ref[...], preferred_element_type=jnp.float32) + bc_ref[...]
    out_ref[...] = 1.0 / (1.0 + jnp.exp(-z))


def _mlp(x, w1, b1, w2, b2, wc, bc):
    return pl.pallas_call(
        _mlp_body,
        out_shape=jax.ShapeDtypeStruct((_B, _PAD_T), jnp.float32),
    )(x, w1, b1, w2, b2, wc, bc)


def kernel(inputs, table, W1, b1, W2, b2, Wc, bc):
    idx_flat = inputs.reshape(-1).astype(jnp.int32)
    tail = table[_NFULL * _VB:].reshape(-1)
    lin = _get_transpose()(table.T, tail)
    pooled = _get_pool()(idx_flat, lin.reshape(_VOCAB, _D))
    wc_p = jnp.pad(Wc, ((0, 0), (0, _PAD_T - _TOPICS)))
    bc_p = jnp.pad(bc, (0, _PAD_T - _TOPICS))
    out = _mlp(pooled, W1, b1.reshape(1, -1), W2, b2.reshape(1, -1),
               wc_p, bc_p.reshape(1, -1))
    return out[:, :_TOPICS]


# speedup vs baseline: 1.6882x; 1.6882x over previous
import functools

import jax
import jax.numpy as jnp
from jax import lax
from jax.experimental import pallas as pl
from jax.experimental.pallas import tpu as pltpu
from jax.experimental.pallas import tpu_sc as plsc

_VOCAB = 1000000
_L = 200
_B = 4096
_D = 64
_TOPICS = 50
_PAD_T = 128

_NC = 2
_NS = 16
_NW = _NC * _NS
_SPW = _B // _NW
_LANES = 16

_VB = 128
_NFULL = _VOCAB // _VB
_TAIL = _VOCAB - _NFULL * _VB
_NIT = 246

_CH0 = 128
_CH1 = _L - _CH0


def _tr_body(tt_hbm, tail_hbm, out_hbm, in0, in1, ob0, ob1, ibuf, si0, si1, so0, so1):
    wid = lax.axis_index("s") * _NC + lax.axis_index("c")
    inbufs = (in0, in1)
    obufs = (ob0, ob1)
    sins = (si0, si1)
    souts = (so0, so1)
    rows = [lax.iota(jnp.int32, 16) + 16 * e for e in range(_D // _LANES)]

    @pl.when(wid == 0)
    def _():
        pltpu.sync_copy(tail_hbm, out_hbm.at[pl.ds(_NFULL * _VB * _D, _TAIL * _D)])

    def blk(it):
        return wid + _NW * it

    def start_in(it, which):
        b = blk(it)

        @pl.when(b < _NFULL)
        def _():
            c0 = pl.multiple_of(b * _VB, _VB)
            pltpu.make_async_copy(
                tt_hbm.at[:, pl.ds(c0, _VB)], inbufs[which], sins[which]).start()

    def wait_in(it, which):
        b = blk(it)

        @pl.when(b < _NFULL)
        def _():
            pltpu.make_async_copy(
                tt_hbm.at[:, pl.ds(0, _VB)], inbufs[which], sins[which]).wait()

    lane = lax.iota(jnp.int32, 16)
    perms = [jnp.bitwise_and(lane + k, 15) for k in range(16)]
    for k in range(16):
        ibuf[pl.ds(16 * k, 16)] = perms[k]
        ibuf[pl.ds(336 + 16 * k, 16)] = perms[k] * _D + lane
    for e in range(_D // _LANES):
        ibuf[pl.ds(272 + 16 * e, 16)] = rows[e]

    def extract(it, which):
        b = blk(it)
        inb = inbufs[which]
        obuf = obufs[which]

        def jbody(jj, _):
            j0 = jj * 16
            for e0 in range(_D // _LANES):
                rvec = ibuf[pl.ds(272 + 16 * e0, 16)]
                sbase = pl.multiple_of(j0 * _D + e0 * 16, 8)
                dst = obuf.at[pl.ds(sbase, 1024)]
                for k0 in range(0, 16, 8):
                    vals = []
                    for k in range(k0, k0 + 8):
                        col = ibuf[pl.ds(16 * k, 16)] + j0
                        vals.append(plsc.load_gather(inb, [rvec, col]))
                    for k, v in zip(range(k0, k0 + 8), vals):
                        dvec = ibuf[pl.ds(336 + 16 * k, 16)]
                        plsc.store_scatter(dst, [dvec], v)
            return 0

        @pl.when(b < _NFULL)
        def _():
            lax.fori_loop(0, _VB // _LANES, jbody, 0)

    def start_out(it, which):
        b = blk(it)

        @pl.when(b < _NFULL)
        def _():
            o0 = pl.multiple_of(b * (_VB * _D), _VB * _D)
            pltpu.make_async_copy(
                obufs[which].at[pl.ds(0, _VB * _D)],
                out_hbm.at[pl.ds(o0, _VB * _D)], souts[which]).start()

    def wait_out(it, which):
        b = blk(it)

        @pl.when(b < _NFULL)
        def _():
            pltpu.make_async_copy(
                obufs[which].at[pl.ds(0, _VB * _D)],
                out_hbm.at[pl.ds(0, _VB * _D)], souts[which]).wait()

    start_in(0, 0)

    def pair(p, _):
        i0 = 2 * p
        start_in(i0 + 1, 1)
        wait_in(i0, 0)

        @pl.when(p > 0)
        def _():
            wait_out(i0 - 2, 0)

        extract(i0, 0)
        start_out(i0, 0)
        start_in(i0 + 2, 0)
        wait_in(i0 + 1, 1)

        @pl.when(p > 0)
        def _():
            wait_out(i0 - 1, 1)

        extract(i0 + 1, 1)
        start_out(i0 + 1, 1)
        return 0

    lax.fori_loop(0, _NIT // 2, pair, 0)
    wait_out(_NIT - 2, 0)
    wait_out(_NIT - 1, 1)


@functools.cache
def _get_transpose():
    return pl.kernel(
        _tr_body,
        out_type=jax.ShapeDtypeStruct((_VOCAB * _D,), jnp.float32),
        mesh=plsc.VectorSubcoreMesh(core_axis_name="c", subcore_axis_name="s"),
        scratch_types=[
            pltpu.VMEM((_D, _VB), jnp.float32),
            pltpu.VMEM((_D, _VB), jnp.float32),
            pltpu.VMEM((_VB * _D + 64,), jnp.float32),
            pltpu.VMEM((_VB * _D + 64,), jnp.float32),
            pltpu.VMEM((592,), jnp.int32),
            pltpu.SemaphoreType.DMA,
            pltpu.SemaphoreType.DMA,
            pltpu.SemaphoreType.DMA,
            pltpu.SemaphoreType.DMA,
        ],
        compiler_params=pltpu.CompilerParams(needs_layout_passes=False),
    )


def _pool_body(idx_hbm, table_hbm, out_hbm, idx_v, buf0, buf1, sums_v, sem0, sem1):
    wid = lax.axis_index("s") * _NC + lax.axis_index("c")
    base = wid * _SPW
    pltpu.sync_copy(idx_hbm.at[pl.ds(base * _L, _SPW * _L)], idx_v)

    bufs = (buf0, buf1)
    sems = (sem0, sem1)

    def start_gather(s, which):
        buf = bufs[which]
        sem = sems[which]
        off = pl.multiple_of(s * _L, 8)
        c0 = pltpu.make_async_copy(
            table_hbm.at[idx_v.at[pl.ds(off, _CH0)]], buf.at[pl.ds(0, _CH0)], sem)
        c1 = pltpu.make_async_copy(
            table_hbm.at[idx_v.at[pl.ds(off + _CH0, _CH1)]], buf.at[pl.ds(_CH0, _CH1)], sem)
        c0.start()
        c1.start()

    def wait_gather(which):
        buf = bufs[which]
        sem = sems[which]
        pltpu.make_async_copy(
            table_hbm.at[idx_v.at[pl.ds(0, _CH0)]], buf.at[pl.ds(0, _CH0)], sem).wait()
        pltpu.make_async_copy(
            table_hbm.at[idx_v.at[pl.ds(0, _CH1)]], buf.at[pl.ds(_CH0, _CH1)], sem).wait()

    def accumulate(s, which):
        buf = bufs[which]

        def rbody(r, accs):
            return tuple(a + buf[r, pl.ds(c * _LANES, _LANES)]
                         for c, a in enumerate(accs))

        accs = lax.fori_loop(
            0, _L, rbody,
            tuple(jnp.zeros((_LANES,), jnp.float32) for _ in range(_D // _LANES)))
        for c, a in enumerate(accs):
            sums_v[s, pl.ds(c * _LANES, _LANES)] = a * (1.0 / _L)

    start_gather(0, 0)

    def pair_body(p, _):
        s0 = p * 2
        start_gather(s0 + 1, 1)
        wait_gather(0)
        accumulate(s0, 0)

        @pl.when(s0 + 2 < _SPW)
        def _():
            start_gather(s0 + 2, 0)

        wait_gather(1)
        accumulate(s0 + 1, 1)
        return 0

    lax.fori_loop(0, _SPW // 2, pair_body, 0)
    pltpu.sync_copy(sums_v, out_hbm.at[pl.ds(base, _SPW)])


@functools.cache
def _get_pool():
    return pl.kernel(
        _pool_body,
        out_type=jax.ShapeDtypeStruct((_B, _D), jnp.float32),
        mesh=plsc.VectorSubcoreMesh(core_axis_name="c", subcore_axis_name="s"),
        scratch_types=[
            pltpu.VMEM((_SPW * _L,), jnp.int32),
            pltpu.VMEM((_L, _D), jnp.float32),
            pltpu.VMEM((_L, _D), jnp.float32),
            pltpu.VMEM((_SPW, _D), jnp.float32),
            pltpu.SemaphoreType.DMA,
            pltpu.SemaphoreType.DMA,
        ],
        compiler_params=pltpu.CompilerParams(use_tc_tiling_on_sc=False),
    )


def _mlp_body(x_ref, w1_ref, b1_ref, w2_ref, b2_ref, wc_ref, bc_ref, out_ref):
    x = x_ref[...]
    h = jnp.maximum(jnp.dot(x, w1_ref[...],
                            preferred_element_type=jnp.float32) + b1_ref[...], 0.0)
    h = jnp.maximum(jnp.dot(h, w2_ref[...],
                            preferred_element_type=jnp.float32) + b2_ref[...], 0.0)
    z = jnp.dot(h, wc_ref[...], preferred_element_type=jnp.float32) + bc_ref[...]
    out_ref[...] = 1.0 / (1.0 + jnp.exp(-z))


def _mlp(x, w1, b1, w2, b2, wc, bc):
    return pl.pallas_call(
        _mlp_body,
        out_shape=jax.ShapeDtypeStruct((_B, _PAD_T), jnp.float32),
    )(x, w1, b1, w2, b2, wc, bc)


def kernel(inputs, table, W1, b1, W2, b2, Wc, bc):
    idx_flat = inputs.reshape(-1).astype(jnp.int32)
    tail = table[_NFULL * _VB:].reshape(-1)
    lin = _get_transpose()(table.T, tail)
    pooled = _get_pool()(idx_flat, lin.reshape(_VOCAB, _D))
    wc_p = jnp.pad(Wc, ((0, 0), (0, _PAD_T - _TOPICS)))
    bc_p = jnp.pad(bc, (0, _PAD_T - _TOPICS))
    out = _mlp(pooled, W1, b1.reshape(1, -1), W2, b2.reshape(1, -1),
               wc_p, bc_p.reshape(1, -1))
    return out[:, :_TOPICS]
